# baseline (speedup 1.0000x reference)
import functools

import jax
import jax.numpy as jnp
from jax import lax
from jax.experimental import pallas as pl
from jax.experimental.pallas import tpu as pltpu
from jax.experimental.pallas import tpu_sc as plsc

_NCLS = 256
_NBINS = _NCLS * _NCLS
_P = 32768


def _mlp_body(x_ref, seg_ref, w1_ref, b1_ref, w2_ref, b2_ref, out_ref):
    x = x_ref[0].reshape(3, _P)
    h = lax.dot_general(w1_ref[:], x, (((1,), (0,)), ((), ())),
                        preferred_element_type=jnp.float32)
    h = jnp.maximum(h + b1_ref[:], 0.0)
    logits = lax.dot_general(w2_ref[:], h, (((1,), (0,)), ((), ())),
                             preferred_element_type=jnp.float32)
    logits = logits + b2_ref[:]
    m = jnp.max(logits, axis=0, keepdims=True)
    iota = lax.broadcasted_iota(jnp.int32, logits.shape,
                                0).astype(jnp.float32)
    pred = jnp.min(jnp.where(logits == m, iota, float(_NBINS)),
                   axis=0).astype(jnp.int32)
    gt = (seg_ref[0, 0].reshape(_P) * 255.0).astype(jnp.int32)
    out_ref[:] = pred * _NCLS + gt


def _merge_indices(x, seg, w1, b1, w2, b2):
    n = x.shape[0] * x.shape[2] * x.shape[3]
    rows = _P // 512
    steps_per_img = x.shape[2] // rows
    return pl.pallas_call(
        _mlp_body,
        grid=(n // _P,),
        in_specs=[
            pl.BlockSpec((1, 3, rows, 512),
                         lambda i: (i // steps_per_img, 0,
                                    i % steps_per_img, 0)),
            pl.BlockSpec((1, 1, rows, 512),
                         lambda i: (i // steps_per_img, 0,
                                    i % steps_per_img, 0)),
            pl.BlockSpec((128, 3), lambda i: (0, 0)),
            pl.BlockSpec((128, 1), lambda i: (0, 0)),
            pl.BlockSpec((_NCLS, 128), lambda i: (0, 0)),
            pl.BlockSpec((_NCLS, 1), lambda i: (0, 0)),
        ],
        out_specs=pl.BlockSpec((_P,), lambda i: (i,)),
        out_shape=jax.ShapeDtypeStruct((n,), jnp.int32),
    )(x, seg, w1, b1, w2, b2)


_NR = _NBINS // 128


def _hist_body(merge_hbm, out_hbm, tab_v, chunk_v, idx_v, shared, sem, chunk):
    c = lax.axis_index("c")
    s = lax.axis_index("s")
    row = c * 16 + s
    zeros16 = jnp.zeros((16,), jnp.int32)
    ones16 = jnp.ones((16,), jnp.int32)
    iota16 = lax.iota(jnp.int32, 16)

    @plsc.parallel_loop(0, _NBINS // 16, unroll=8)
    def zbody(i):
        tab_v[i // 8, pl.ds((i % 8) * 16, 16)] = zeros16

    @pl.when(s == 0)
    def _():
        pltpu.sync_copy(tab_v, shared)

    def ibody(i, _):
        idx_v[i // 8, pl.ds((i % 8) * 16, 16)] = i * 16 + iota16
        return 0

    lax.fori_loop(0, _NR // 16, ibody, 0, unroll=4)

    pltpu.sync_copy(merge_hbm.at[pl.ds(row * chunk, chunk)], chunk_v)

    @plsc.parallel_loop(0, chunk // 16, unroll=8)
    def sbody(i):
        idx = chunk_v[pl.ds(i * 16, 16)]
        plsc.addupdate_scatter(tab_v, [idx >> 7, idx & 127], ones16)

    plsc.subcore_barrier()
    descs = []
    for j in range(_NR // 128):
        descs.append(pltpu.async_copy(
            tab_v.at[pl.ds(j * 128, 128), :], shared.at[idx_v.at[j]], sem,
            add=True))
    for d in descs:
        d.wait()
    plsc.subcore_barrier()

    sl = _NR // 16
    pltpu.sync_copy(shared.at[pl.ds(s * sl, sl), :],
                    out_hbm.at[c, pl.ds(s * sl, sl), :])


def _histogram_partials(merge):
    n = merge.shape[0]
    chunk = n // 32
    mesh = plsc.VectorSubcoreMesh(core_axis_name="c", subcore_axis_name="s")
    body = functools.partial(_hist_body, chunk=chunk)
    f = pl.kernel(
        body,
        out_type=jax.ShapeDtypeStruct((2, _NR, 128), jnp.int32),
        mesh=mesh,
        compiler_params=pltpu.CompilerParams(needs_layout_passes=False),
        scratch_types=[
            pltpu.VMEM((_NR, 128), jnp.int32),
            pltpu.VMEM((chunk,), jnp.int32),
            pltpu.VMEM((_NR // 128, 128), jnp.int32),
            pltpu.VMEM_SHARED((_NR, 128), jnp.int32),
            pltpu.SemaphoreType.DMA,
        ],
    )
    return f(merge).reshape(2, _NBINS)


def kernel(fake_images, segmaps, W1, b1, W2, b2):
    B, C, H, W = fake_images.shape
    merge = _merge_indices(fake_images, segmaps, W1, b1[:, None], W2,
                           b2[:, None])
    hist = _histogram_partials(merge)
    return hist.reshape(B, _NCLS, _NCLS)
